# baseline (speedup 1.0000x reference)
import functools

import jax
import jax.numpy as jnp
from jax import lax
from jax.experimental import pallas as pl
from jax.experimental.pallas import tpu as pltpu
from jax.experimental.pallas import tpu_sc as plsc

N = 10000
E = 320000
D = 128

NC = 2
NS = 16
NW = NC * NS
EPW = E // NW
B = 80
NB = EPW // B
NBUF = 4
LA = 3
IBUF = 8
LAI = 6
ROWS_PT = 624
REM_ROWS = N - NS * ROWS_PT
LANES = 16


def _mmerge_body(a_ref, b_ref, w_ref, o_ref):
    o_ref[...] = jnp.dot(a_ref[...] + b_ref[...], w_ref[...],
                         preferred_element_type=jnp.float32)


def _tc_merge_matmul(a, b, w):
    g = 10
    return pl.pallas_call(
        _mmerge_body,
        grid=(g,),
        in_specs=[
            pl.BlockSpec((N // g, D), lambda i: (i, 0)),
            pl.BlockSpec((N // g, D), lambda i: (i, 0)),
            pl.BlockSpec((D, D), lambda i: (0, 0)),
        ],
        out_specs=pl.BlockSpec((N // g, D), lambda i: (i, 0)),
        out_shape=jax.ShapeDtypeStruct((N, D), jnp.float32),
    )(a, b, w)


@functools.partial(
    pl.kernel,
    out_type=jax.ShapeDtypeStruct((NC, N, D), jnp.float32),
    mesh=plsc.VectorSubcoreMesh(core_axis_name="c", subcore_axis_name="s"),
    scratch_types=[
        pltpu.VMEM((NBUF * B, D), jnp.float32),
        pltpu.VMEM((IBUF, B), jnp.int32),
        pltpu.VMEM((IBUF, B), jnp.int32),
        pltpu.VMEM((IBUF, B), jnp.float32),
        pltpu.VMEM_SHARED((N, D), jnp.float32),
        pltpu.SemaphoreType.DMA((NBUF,)),
        pltpu.SemaphoreType.DMA((NBUF,)),
        pltpu.SemaphoreType.DMA((IBUF,)),
    ],
)
def _sc_spmm(m_hbm, row_hbm, col_hbm, val_hbm, out_hbm,
             stage, rowr, colr, valr, acc, gsem, ssem, isem):
    c = lax.axis_index("c")
    s = lax.axis_index("s")
    wid = s * NC + c

    def _slot(b):
        return pl.ds(pl.multiple_of(lax.rem(b, NBUF) * B, 8), B)

    def zrow(b, carry):
        for j in range(D // LANES):
            stage[b, pl.ds(j * LANES, LANES)] = jnp.zeros((LANES,), jnp.float32)
        return carry

    lax.fori_loop(0, B, zrow, 0)

    base_r = pl.multiple_of(s * ROWS_PT, 8)
    n_full = ROWS_PT // B
    rem = ROWS_PT - n_full * B

    def zcp(k, carry):
        pltpu.sync_copy(stage.at[pl.ds(0, B)],
                        acc.at[pl.ds(pl.multiple_of(base_r + k * B, 8), B)])
        return carry

    lax.fori_loop(0, n_full, zcp, 0)
    if rem:
        pltpu.sync_copy(stage.at[pl.ds(0, rem)],
                        acc.at[pl.ds(pl.multiple_of(base_r + n_full * B, 8), rem)])

    @pl.when(s == NS - 1)
    def _zero_tail():
        pltpu.sync_copy(stage.at[pl.ds(0, REM_ROWS)],
                        acc.at[pl.ds(NS * ROWS_PT, REM_ROWS)])

    plsc.subcore_barrier()

    def _idx_start(ki, islot):
        pltpu.async_copy(row_hbm.at[wid, ki], rowr.at[islot], isem.at[islot])
        pltpu.async_copy(col_hbm.at[wid, ki], colr.at[islot], isem.at[islot])
        pltpu.async_copy(val_hbm.at[wid, ki], valr.at[islot], isem.at[islot])

    def _idx_wait(ki, islot):
        pltpu.make_async_copy(row_hbm.at[wid, ki], rowr.at[islot],
                              isem.at[islot]).wait()
        pltpu.make_async_copy(col_hbm.at[wid, ki], colr.at[islot],
                              isem.at[islot]).wait()
        pltpu.make_async_copy(val_hbm.at[wid, ki], valr.at[islot],
                              isem.at[islot]).wait()

    for ki in range(LAI):
        _idx_start(ki, ki)
    for kg in range(LA):
        _idx_wait(kg, kg)
        pltpu.async_copy(m_hbm.at[colr.at[kg]], stage.at[_slot(kg)],
                         gsem.at[kg % NBUF])

    def batch(kk, carry):
        sl_cur = _slot(kk)
        bslot = lax.rem(kk, NBUF)
        islot = lax.rem(kk, IBUF)
        pltpu.make_async_copy(m_hbm.at[colr.at[islot]], stage.at[sl_cur],
                              gsem.at[bslot]).wait()
        base = pl.multiple_of(bslot * B, 8)

        def egroup(g, carry2):
            vals16 = valr[islot, pl.ds(g * LANES, LANES)]
            for i in range(LANES):
                vv = lax.gather(
                    vals16, jnp.full((LANES, 1), i, jnp.int32),
                    lax.GatherDimensionNumbers(offset_dims=(),
                                               collapsed_slice_dims=(0,),
                                               start_index_map=(0,)),
                    slice_sizes=(1,),
                    mode=lax.GatherScatterMode.PROMISE_IN_BOUNDS)
                e = base + g * LANES + i
                for j in range(D // LANES):
                    sl = pl.ds(j * LANES, LANES)
                    stage[e, sl] = stage[e, sl] * vv
            return carry2

        for _g in range(B // LANES):
            egroup(_g, 0)

        @pl.when(kk >= 1)
        def _wait_prev_scatter():
            pslot = lax.rem(kk - 1, NBUF)
            pltpu.make_async_copy(stage.at[pl.ds(pl.multiple_of(pslot * B, 8), B)],
                                  acc.at[rowr.at[islot]],
                                  ssem.at[pslot]).wait()

        pltpu.async_copy(stage.at[sl_cur], acc.at[rowr.at[islot]],
                         ssem.at[bslot], add=True)

        kg = kk + LA

        @pl.when(kg < NB)
        def _prefetch_gather():
            gslot = lax.rem(kg, NBUF)
            gisl = lax.rem(kg, IBUF)
            _idx_wait(kg, gisl)
            pltpu.async_copy(m_hbm.at[colr.at[gisl]], stage.at[_slot(kg)],
                             gsem.at[gslot])

        ki = kk + LAI

        @pl.when(ki < NB)
        def _prefetch_idx():
            _idx_start(ki, lax.rem(ki, IBUF))

        return carry

    lax.fori_loop(0, NB, batch, 0)

    kkt = NB - 1
    pltpu.make_async_copy(stage.at[_slot(kkt)], acc.at[rowr.at[kkt % IBUF]],
                          ssem.at[kkt % NBUF]).wait()

    plsc.subcore_barrier()
    pltpu.sync_copy(acc.at[pl.ds(base_r, ROWS_PT)],
                    out_hbm.at[c, pl.ds(base_r, ROWS_PT)])

    @pl.when(s == NS - 1)
    def _write_tail():
        pltpu.sync_copy(acc.at[pl.ds(NS * ROWS_PT, REM_ROWS)],
                        out_hbm.at[c, pl.ds(NS * ROWS_PT, REM_ROWS)])


def kernel(x, edge_index, edge_val, weight):
    row3 = edge_index[0].reshape(NW, NB, B)
    col3 = edge_index[1].reshape(NW, NB, B)
    val3 = edge_val.reshape(NW, NB, B)
    parts = _sc_spmm(x, row3, col3, val3)
    return _tc_merge_matmul(parts[0], parts[1], weight)


# speedup vs baseline: 1.0281x; 1.0281x over previous
import functools

import jax
import jax.numpy as jnp
from jax import lax
from jax.experimental import pallas as pl
from jax.experimental.pallas import tpu as pltpu
from jax.experimental.pallas import tpu_sc as plsc

N = 10000
E = 320000
D = 128

NC = 2
NS = 16
NW = NC * NS
EPW = E // NW
B = 80
NB = EPW // B
NBUF = 4
LA = 3
IBUF = 8
LAI = 6
ROWS_PT = 624
REM_ROWS = N - NS * ROWS_PT
LANES = 16


def _mmerge_body(a_ref, b_ref, w_ref, o_ref):
    o_ref[...] = jnp.dot(a_ref[...] + b_ref[...], w_ref[...],
                         preferred_element_type=jnp.float32)


def _tc_merge_matmul(a, b, w):
    g = 10
    return pl.pallas_call(
        _mmerge_body,
        grid=(g,),
        in_specs=[
            pl.BlockSpec((N // g, D), lambda i: (i, 0)),
            pl.BlockSpec((N // g, D), lambda i: (i, 0)),
            pl.BlockSpec((D, D), lambda i: (0, 0)),
        ],
        out_specs=pl.BlockSpec((N // g, D), lambda i: (i, 0)),
        out_shape=jax.ShapeDtypeStruct((N, D), jnp.float32),
    )(a, b, w)


@functools.partial(
    pl.kernel,
    out_type=jax.ShapeDtypeStruct((NC, N, D), jnp.float32),
    mesh=plsc.VectorSubcoreMesh(core_axis_name="c", subcore_axis_name="s"),
    scratch_types=[
        pltpu.VMEM((NBUF * B, D), jnp.float32),
        pltpu.VMEM((IBUF, B), jnp.int32),
        pltpu.VMEM((IBUF, B), jnp.int32),
        pltpu.VMEM((IBUF, B), jnp.float32),
        pltpu.VMEM_SHARED((N, D), jnp.float32),
        pltpu.SemaphoreType.DMA((NBUF,)),
        pltpu.SemaphoreType.DMA((NBUF,)),
        pltpu.SemaphoreType.DMA((IBUF,)),
    ],
)
def _sc_spmm(m_hbm, row_hbm, col_hbm, val_hbm, out_hbm,
             stage, rowr, colr, valr, acc, gsem, ssem, isem):
    c = lax.axis_index("c")
    s = lax.axis_index("s")
    wid = s * NC + c

    def _slot(b):
        return pl.ds(pl.multiple_of(lax.rem(b, NBUF) * B, 8), B)

    def _idx_start(ki, islot):
        pltpu.async_copy(row_hbm.at[wid, ki], rowr.at[islot], isem.at[islot])
        pltpu.async_copy(col_hbm.at[wid, ki], colr.at[islot], isem.at[islot])
        pltpu.async_copy(val_hbm.at[wid, ki], valr.at[islot], isem.at[islot])

    def _idx_wait(ki, islot):
        pltpu.make_async_copy(row_hbm.at[wid, ki], rowr.at[islot],
                              isem.at[islot]).wait()
        pltpu.make_async_copy(col_hbm.at[wid, ki], colr.at[islot],
                              isem.at[islot]).wait()
        pltpu.make_async_copy(val_hbm.at[wid, ki], valr.at[islot],
                              isem.at[islot]).wait()

    for ki in range(LAI):
        _idx_start(ki, ki)
    for kg in range(LA):
        _idx_wait(kg, kg)
        pltpu.async_copy(m_hbm.at[colr.at[kg]], stage.at[_slot(kg)],
                         gsem.at[kg % NBUF])

    ZSLOT = (NBUF - 1) * B

    def zrow(b, carry):
        for j in range(D // LANES):
            stage[ZSLOT + b, pl.ds(j * LANES, LANES)] = jnp.zeros(
                (LANES,), jnp.float32)
        return carry

    lax.fori_loop(0, B, zrow, 0)

    base_r = pl.multiple_of(s * ROWS_PT, 8)
    n_full = ROWS_PT // B
    rem = ROWS_PT - n_full * B

    def zcp(k, carry):
        pltpu.sync_copy(stage.at[pl.ds(ZSLOT, B)],
                        acc.at[pl.ds(pl.multiple_of(base_r + k * B, 8), B)])
        return carry

    lax.fori_loop(0, n_full, zcp, 0)
    if rem:
        pltpu.sync_copy(stage.at[pl.ds(ZSLOT, rem)],
                        acc.at[pl.ds(pl.multiple_of(base_r + n_full * B, 8), rem)])

    @pl.when(s == NS - 1)
    def _zero_tail():
        pltpu.sync_copy(stage.at[pl.ds(ZSLOT, REM_ROWS)],
                        acc.at[pl.ds(NS * ROWS_PT, REM_ROWS)])

    plsc.subcore_barrier()

    def batch(kk, carry):
        sl_cur = _slot(kk)
        bslot = lax.rem(kk, NBUF)
        islot = lax.rem(kk, IBUF)
        pltpu.make_async_copy(m_hbm.at[colr.at[islot]], stage.at[sl_cur],
                              gsem.at[bslot]).wait()
        base = pl.multiple_of(bslot * B, 8)

        def egroup(g, carry2):
            vals16 = valr[islot, pl.ds(g * LANES, LANES)]
            for i in range(LANES):
                vv = lax.gather(
                    vals16, jnp.full((LANES, 1), i, jnp.int32),
                    lax.GatherDimensionNumbers(offset_dims=(),
                                               collapsed_slice_dims=(0,),
                                               start_index_map=(0,)),
                    slice_sizes=(1,),
                    mode=lax.GatherScatterMode.PROMISE_IN_BOUNDS)
                e = base + g * LANES + i
                for j in range(D // LANES):
                    sl = pl.ds(j * LANES, LANES)
                    stage[e, sl] = stage[e, sl] * vv
            return carry2

        for _g in range(B // LANES):
            egroup(_g, 0)

        @pl.when(kk >= 1)
        def _wait_prev_scatter():
            pslot = lax.rem(kk - 1, NBUF)
            pltpu.make_async_copy(stage.at[pl.ds(pl.multiple_of(pslot * B, 8), B)],
                                  acc.at[rowr.at[islot]],
                                  ssem.at[pslot]).wait()

        pltpu.async_copy(stage.at[sl_cur], acc.at[rowr.at[islot]],
                         ssem.at[bslot], add=True)

        kg = kk + LA

        @pl.when(kg < NB)
        def _prefetch_gather():
            gslot = lax.rem(kg, NBUF)
            gisl = lax.rem(kg, IBUF)
            _idx_wait(kg, gisl)
            pltpu.async_copy(m_hbm.at[colr.at[gisl]], stage.at[_slot(kg)],
                             gsem.at[gslot])

        ki = kk + LAI

        @pl.when(ki < NB)
        def _prefetch_idx():
            _idx_start(ki, lax.rem(ki, IBUF))

        return carry

    lax.fori_loop(0, NB, batch, 0)

    kkt = NB - 1
    pltpu.make_async_copy(stage.at[_slot(kkt)], acc.at[rowr.at[kkt % IBUF]],
                          ssem.at[kkt % NBUF]).wait()

    plsc.subcore_barrier()
    pltpu.sync_copy(acc.at[pl.ds(base_r, ROWS_PT)],
                    out_hbm.at[c, pl.ds(base_r, ROWS_PT)])

    @pl.when(s == NS - 1)
    def _write_tail():
        pltpu.sync_copy(acc.at[pl.ds(NS * ROWS_PT, REM_ROWS)],
                        out_hbm.at[c, pl.ds(NS * ROWS_PT, REM_ROWS)])


def kernel(x, edge_index, edge_val, weight):
    row3 = edge_index[0].reshape(NW, NB, B)
    col3 = edge_index[1].reshape(NW, NB, B)
    val3 = edge_val.reshape(NW, NB, B)
    parts = _sc_spmm(x, row3, col3, val3)
    return _tc_merge_matmul(parts[0], parts[1], weight)
